# baseline (speedup 1.0000x reference)
import functools

import jax
import jax.numpy as jnp
from jax import lax
from jax.experimental import pallas as pl
from jax.experimental.pallas import tpu as pltpu
from jax.experimental.pallas import tpu_sc as plsc

NC = 2
NS = 16
CH = 128


def _make_seg_sum(npad, d, epad, f0=0.5):
    nw = NC * NS
    total_chunks = epad // CH
    nch0 = int(round(total_chunks * f0 / (NS * 8))) * 8
    nch0 = max(8, min(nch0, total_chunks // NS - 8))
    nch1 = total_chunks // NS - nch0
    c0_rows = NS * nch0
    rps = npad // NS
    mesh = plsc.VectorSubcoreMesh(core_axis_name="c", subcore_axis_name="s")

    @functools.partial(
        pl.kernel,
        mesh=mesh,
        out_type=[
            jax.ShapeDtypeStruct((NC, npad, d), jnp.float32),
            jax.ShapeDtypeStruct((NC, npad, d), jnp.float32),
            jax.ShapeDtypeStruct((npad, d), jnp.float32),
        ],
        scratch_types=[
            pltpu.VMEM_SHARED((npad, d), jnp.float32),
            pltpu.VMEM((8, CH), jnp.int32),
            pltpu.VMEM((8, CH), jnp.int32),
            pltpu.VMEM((2, CH, d), jnp.float32),
            pltpu.SemaphoreType.DMA,
            pltpu.SemaphoreType.DMA,
            pltpu.SemaphoreType.DMA,
            pltpu.SemaphoreType.DMA,
        ],
    )
    def seg_sum(x_hbm, h_hbm, src_hbm, dst_hbm, zr_hbm, on_hbm,
                px_hbm, ph_hbm, pc_hbm,
                acc, src_v, dst_v, rows_v, gs0, gs1, ss0, ss1):
        cid = lax.axis_index("c")
        sid = lax.axis_index("s")
        own = pl.ds(sid * rps, rps)
        gsem = (gs0, gs1)
        ssem = (ss0, ss1)
        wbase = jnp.where(cid == 0, sid * nch0, c0_rows + sid * nch1)
        ngroups = jnp.where(cid == 0, nch0 // 8, nch1 // 8)

        def zero_acc():
            pltpu.sync_copy(zr_hbm, acc.at[own])

        def dump(out_hbm):
            pltpu.sync_copy(acc.at[own], out_hbm.at[cid].at[own])

        def cnt_phase():
            pltpu.sync_copy(on_hbm, rows_v.at[0])

            def group(g, carry):
                base = sid * (total_chunks // NS) + g * 8
                pltpu.sync_copy(dst_hbm.at[pl.ds(base, 8)], dst_v)
                sca = []
                for k in range(8):
                    if k >= 2:
                        sca[k - 2].wait()
                    sca.append(pltpu.async_copy(
                        rows_v.at[0], acc.at[dst_v.at[k]],
                        ssem[k % 2], add=True))
                sca[6].wait()
                sca[7].wait()
                return carry

            lax.fori_loop(0, total_chunks // (NS * 8), group, 0)

        def agg_phase(table_hbm):
            def group(g, carry):
                base = wbase + g * 8
                pltpu.sync_copy(src_hbm.at[pl.ds(base, 8)], src_v)
                pltpu.sync_copy(dst_hbm.at[pl.ds(base, 8)], dst_v)
                gat = [pltpu.async_copy(
                    table_hbm.at[src_v.at[0]], rows_v.at[0], gsem[0])]
                sca = []
                for k in range(8):
                    gat[k].wait()
                    sca.append(pltpu.async_copy(
                        rows_v.at[k % 2], acc.at[dst_v.at[k]],
                        ssem[k % 2], add=True))
                    if k < 7:
                        if k >= 1:
                            sca[k - 1].wait()
                        gat.append(pltpu.async_copy(
                            table_hbm.at[src_v.at[k + 1]],
                            rows_v.at[(k + 1) % 2], gsem[(k + 1) % 2]))
                sca[6].wait()
                sca[7].wait()
                return carry

            lax.fori_loop(0, ngroups, group, 0)

        @pl.when(cid == 1)
        def _():
            zero_acc()
            plsc.subcore_barrier()
            cnt_phase()
            plsc.subcore_barrier()
            pltpu.sync_copy(acc.at[own], pc_hbm.at[own])

        zero_acc()
        plsc.subcore_barrier()
        agg_phase(x_hbm)
        plsc.subcore_barrier()
        dump(px_hbm)

        zero_acc()
        plsc.subcore_barrier()
        agg_phase(h_hbm)
        plsc.subcore_barrier()
        dump(ph_hbm)

    return seg_sum


def _tc_body(px, ph, pc, x, h, c, w, b, wi, wf, wo, bi, bf, bc, bo,
             o_ref, hn_ref, cn_ref):
    sx = px[0] + px[1]
    sh = ph[0] + ph[1]
    cnt = jnp.maximum(pc[:, 0:1], 1.0)
    agg_x = sx / cnt
    agg_h = sh / cnt
    a = jnp.concatenate([agg_x, x[...], agg_h, h[...]], axis=1)
    z = jnp.dot(a, w[...], preferred_element_type=jnp.float32) + b[...]
    hd = x.shape[1]
    c0 = c[...]
    zi = z[:, 0 * hd:1 * hd]
    zf = z[:, 1 * hd:2 * hd]
    zc = z[:, 2 * hd:3 * hd]
    zo = z[:, 3 * hd:4 * hd]
    gi = jax.nn.sigmoid(zi + wi[...] * c0 + bi[...])
    gf = jax.nn.sigmoid(zf + wf[...] * c0 + bf[...])
    cn = gf * c0 + gi * jnp.tanh(zc + bc[...])
    go = jax.nn.sigmoid(zo + wo[...] * cn + bo[...])
    o_ref[...] = go
    hn_ref[...] = go * jnp.tanh(cn)
    cn_ref[...] = cn


def kernel(X, edge_index, h, c,
           ix_Wl, ix_bl, ix_Wr, ih_Wl, ih_bl, ih_Wr,
           fx_Wl, fx_bl, fx_Wr, fh_Wl, fh_bl, fh_Wr,
           cx_Wl, cx_bl, cx_Wr, ch_Wl, ch_bl, ch_Wr,
           ox_Wl, ox_bl, ox_Wr, oh_Wl, oh_bl, oh_Wr,
           w_i, w_f, w_o, b_i, b_f, b_c, b_o):
    n, d = X.shape
    e = edge_index.shape[1]
    hd = h.shape[1]
    nw = NC * NS
    npad = -(-n // (8 * NS)) * (8 * NS)
    rps = npad // NS
    estep = nw * CH * 8
    epad = -(-e // estep) * estep

    src = edge_index[0]
    dst = edge_index[1]
    if epad != e:
        src = jnp.concatenate([src, jnp.zeros((epad - e,), jnp.int32)])
        dst = jnp.concatenate(
            [dst, jnp.full((epad - e,), npad - 1, jnp.int32)])
    srcm = src.reshape(-1, CH)
    dstm = dst.reshape(-1, CH)
    zr = jnp.zeros((rps, d), jnp.float32)
    on = jnp.ones((CH, d), jnp.float32)

    px, ph, pc = _make_seg_sum(npad, d, epad, f0=0.90)(X, h, srcm, dstm, zr, on)

    wcols = []
    for wl_x, wr_x, wl_h, wr_h in (
        (ix_Wl, ix_Wr, ih_Wl, ih_Wr),
        (fx_Wl, fx_Wr, fh_Wl, fh_Wr),
        (cx_Wl, cx_Wr, ch_Wl, ch_Wr),
        (ox_Wl, ox_Wr, oh_Wl, oh_Wr),
    ):
        wcols.append(jnp.concatenate([wl_x.T, wr_x.T, wl_h.T, wr_h.T], axis=0))
    wstack = jnp.concatenate(wcols, axis=1)
    bias = jnp.concatenate(
        [ix_bl + ih_bl, fx_bl + fh_bl, cx_bl + ch_bl, ox_bl + oh_bl]
    ).reshape(1, 4 * hd)

    blk = 400
    grid = (n // blk,)
    full = lambda i: (0, 0)
    row = lambda i: (i, 0)
    part3 = lambda i: (0, i, 0)
    part2 = lambda i: (0, i)
    o, hn, cn = pl.pallas_call(
        _tc_body,
        grid=grid,
        in_specs=[
            pl.BlockSpec((NC, blk, d), part3),
            pl.BlockSpec((NC, blk, d), part3),
            pl.BlockSpec((blk, d), row),
            pl.BlockSpec((blk, d), row),
            pl.BlockSpec((blk, hd), row),
            pl.BlockSpec((blk, hd), row),
            pl.BlockSpec((2 * d + 2 * hd, 4 * hd), full),
            pl.BlockSpec((1, 4 * hd), full),
            pl.BlockSpec((1, hd), full),
            pl.BlockSpec((1, hd), full),
            pl.BlockSpec((1, hd), full),
            pl.BlockSpec((1, hd), full),
            pl.BlockSpec((1, hd), full),
            pl.BlockSpec((1, hd), full),
            pl.BlockSpec((1, hd), full),
        ],
        out_specs=[
            pl.BlockSpec((blk, hd), row),
            pl.BlockSpec((blk, hd), row),
            pl.BlockSpec((blk, hd), row),
        ],
        out_shape=[
            jax.ShapeDtypeStruct((n, hd), jnp.float32),
            jax.ShapeDtypeStruct((n, hd), jnp.float32),
            jax.ShapeDtypeStruct((n, hd), jnp.float32),
        ],
    )(px, ph, pc, X, h, c, wstack, bias, w_i, w_f, w_o, b_i, b_f, b_c, b_o)

    return (o, (hn, cn))


# speedup vs baseline: 1.0126x; 1.0126x over previous
import functools

import jax
import jax.numpy as jnp
from jax import lax
from jax.experimental import pallas as pl
from jax.experimental.pallas import tpu as pltpu
from jax.experimental.pallas import tpu_sc as plsc

NC = 2
NS = 16
CH = 128


def _make_seg_sum(npad, d, epad, f0=0.5):
    nw = NC * NS
    total_chunks = epad // CH
    nch0 = int(round(total_chunks * f0 / (NS * 8))) * 8
    nch0 = max(8, min(nch0, total_chunks // NS - 8))
    nch1 = total_chunks // NS - nch0
    c0_rows = NS * nch0
    rps = npad // NS
    mesh = plsc.VectorSubcoreMesh(core_axis_name="c", subcore_axis_name="s")

    @functools.partial(
        pl.kernel,
        mesh=mesh,
        out_type=[
            jax.ShapeDtypeStruct((NC, npad, d), jnp.float32),
            jax.ShapeDtypeStruct((NC, npad, d), jnp.float32),
            jax.ShapeDtypeStruct((npad, d), jnp.float32),
        ],
        scratch_types=[
            pltpu.VMEM_SHARED((npad, d), jnp.float32),
            pltpu.VMEM((8, CH), jnp.int32),
            pltpu.VMEM((8, CH), jnp.int32),
            pltpu.VMEM((2, CH, d), jnp.float32),
            pltpu.SemaphoreType.DMA,
            pltpu.SemaphoreType.DMA,
            pltpu.SemaphoreType.DMA,
            pltpu.SemaphoreType.DMA,
        ],
    )
    def seg_sum(x_hbm, h_hbm, src_hbm, dst_hbm, zr_hbm, on_hbm,
                px_hbm, ph_hbm, pc_hbm,
                acc, src_v, dst_v, rows_v, gs0, gs1, ss0, ss1):
        cid = lax.axis_index("c")
        sid = lax.axis_index("s")
        own = pl.ds(sid * rps, rps)
        gsem = (gs0, gs1)
        ssem = (ss0, ss1)
        wbase = jnp.where(cid == 0, sid * nch0, c0_rows + sid * nch1)
        ngroups = jnp.where(cid == 0, nch0 // 8, nch1 // 8)

        def zero_acc():
            pltpu.sync_copy(zr_hbm, acc.at[own])

        def dump(out_hbm):
            pltpu.sync_copy(acc.at[own], out_hbm.at[cid].at[own])

        def cnt_phase():
            pltpu.sync_copy(on_hbm, rows_v.at[0])

            def group(g, carry):
                base = sid * (total_chunks // NS) + g * 8
                pltpu.sync_copy(dst_hbm.at[pl.ds(base, 8)], dst_v)
                sca = []
                for k in range(8):
                    if k >= 2:
                        sca[k - 2].wait()
                    sca.append(pltpu.async_copy(
                        rows_v.at[0], acc.at[dst_v.at[k]],
                        ssem[k % 2], add=True))
                sca[6].wait()
                sca[7].wait()
                return carry

            lax.fori_loop(0, total_chunks // (NS * 8), group, 0)

        def agg_phase(table_hbm):
            def group(g, carry):
                base = wbase + g * 8
                pltpu.sync_copy(src_hbm.at[pl.ds(base, 8)], src_v)
                pltpu.sync_copy(dst_hbm.at[pl.ds(base, 8)], dst_v)
                gat = [pltpu.async_copy(
                    table_hbm.at[src_v.at[0]], rows_v.at[0], gsem[0])]
                sca = []
                for k in range(8):
                    gat[k].wait()
                    sca.append(pltpu.async_copy(
                        rows_v.at[k % 2], acc.at[dst_v.at[k]],
                        ssem[k % 2], add=True))
                    if k < 7:
                        if k >= 1:
                            sca[k - 1].wait()
                        gat.append(pltpu.async_copy(
                            table_hbm.at[src_v.at[k + 1]],
                            rows_v.at[(k + 1) % 2], gsem[(k + 1) % 2]))
                sca[6].wait()
                sca[7].wait()
                return carry

            lax.fori_loop(0, ngroups, group, 0)

        @pl.when(cid == 1)
        def _():
            zero_acc()
            plsc.subcore_barrier()
            cnt_phase()
            plsc.subcore_barrier()
            pltpu.sync_copy(acc.at[own], pc_hbm.at[own])

        zero_acc()
        plsc.subcore_barrier()
        agg_phase(x_hbm)
        plsc.subcore_barrier()
        dump(px_hbm)

        zero_acc()
        plsc.subcore_barrier()
        agg_phase(h_hbm)
        plsc.subcore_barrier()
        dump(ph_hbm)

    return seg_sum


def _tc_body(px, ph, pc, x, h, c, w, b, wi, wf, wo, bi, bf, bc, bo,
             o_ref, hn_ref, cn_ref):
    sx = px[0] + px[1]
    sh = ph[0] + ph[1]
    cnt = jnp.maximum(pc[:, 0:1], 1.0)
    agg_x = sx / cnt
    agg_h = sh / cnt
    a = jnp.concatenate([agg_x, x[...], agg_h, h[...]], axis=1)
    z = jnp.dot(a, w[...], preferred_element_type=jnp.float32) + b[...]
    hd = x.shape[1]
    c0 = c[...]
    zi = z[:, 0 * hd:1 * hd]
    zf = z[:, 1 * hd:2 * hd]
    zc = z[:, 2 * hd:3 * hd]
    zo = z[:, 3 * hd:4 * hd]
    gi = jax.nn.sigmoid(zi + wi[...] * c0 + bi[...])
    gf = jax.nn.sigmoid(zf + wf[...] * c0 + bf[...])
    cn = gf * c0 + gi * jnp.tanh(zc + bc[...])
    go = jax.nn.sigmoid(zo + wo[...] * cn + bo[...])
    o_ref[...] = go
    hn_ref[...] = go * jnp.tanh(cn)
    cn_ref[...] = cn


def kernel(X, edge_index, h, c,
           ix_Wl, ix_bl, ix_Wr, ih_Wl, ih_bl, ih_Wr,
           fx_Wl, fx_bl, fx_Wr, fh_Wl, fh_bl, fh_Wr,
           cx_Wl, cx_bl, cx_Wr, ch_Wl, ch_bl, ch_Wr,
           ox_Wl, ox_bl, ox_Wr, oh_Wl, oh_bl, oh_Wr,
           w_i, w_f, w_o, b_i, b_f, b_c, b_o):
    n, d = X.shape
    e = edge_index.shape[1]
    hd = h.shape[1]
    nw = NC * NS
    npad = -(-n // (8 * NS)) * (8 * NS)
    rps = npad // NS
    estep = nw * CH * 8
    epad = -(-e // estep) * estep

    src = edge_index[0]
    dst = edge_index[1]
    if epad != e:
        src = jnp.concatenate([src, jnp.zeros((epad - e,), jnp.int32)])
        dst = jnp.concatenate(
            [dst, jnp.full((epad - e,), npad - 1, jnp.int32)])
    srcm = src.reshape(-1, CH)
    dstm = dst.reshape(-1, CH)
    zr = jnp.zeros((rps, d), jnp.float32)
    on = jnp.ones((CH, d), jnp.float32)

    px, ph, pc = _make_seg_sum(npad, d, epad, f0=0.95)(X, h, srcm, dstm, zr, on)

    wcols = []
    for wl_x, wr_x, wl_h, wr_h in (
        (ix_Wl, ix_Wr, ih_Wl, ih_Wr),
        (fx_Wl, fx_Wr, fh_Wl, fh_Wr),
        (cx_Wl, cx_Wr, ch_Wl, ch_Wr),
        (ox_Wl, ox_Wr, oh_Wl, oh_Wr),
    ):
        wcols.append(jnp.concatenate([wl_x.T, wr_x.T, wl_h.T, wr_h.T], axis=0))
    wstack = jnp.concatenate(wcols, axis=1)
    bias = jnp.concatenate(
        [ix_bl + ih_bl, fx_bl + fh_bl, cx_bl + ch_bl, ox_bl + oh_bl]
    ).reshape(1, 4 * hd)

    blk = 400
    grid = (n // blk,)
    full = lambda i: (0, 0)
    row = lambda i: (i, 0)
    part3 = lambda i: (0, i, 0)
    part2 = lambda i: (0, i)
    o, hn, cn = pl.pallas_call(
        _tc_body,
        grid=grid,
        in_specs=[
            pl.BlockSpec((NC, blk, d), part3),
            pl.BlockSpec((NC, blk, d), part3),
            pl.BlockSpec((blk, d), row),
            pl.BlockSpec((blk, d), row),
            pl.BlockSpec((blk, hd), row),
            pl.BlockSpec((blk, hd), row),
            pl.BlockSpec((2 * d + 2 * hd, 4 * hd), full),
            pl.BlockSpec((1, 4 * hd), full),
            pl.BlockSpec((1, hd), full),
            pl.BlockSpec((1, hd), full),
            pl.BlockSpec((1, hd), full),
            pl.BlockSpec((1, hd), full),
            pl.BlockSpec((1, hd), full),
            pl.BlockSpec((1, hd), full),
            pl.BlockSpec((1, hd), full),
        ],
        out_specs=[
            pl.BlockSpec((blk, hd), row),
            pl.BlockSpec((blk, hd), row),
            pl.BlockSpec((blk, hd), row),
        ],
        out_shape=[
            jax.ShapeDtypeStruct((n, hd), jnp.float32),
            jax.ShapeDtypeStruct((n, hd), jnp.float32),
            jax.ShapeDtypeStruct((n, hd), jnp.float32),
        ],
    )(px, ph, pc, X, h, c, wstack, bias, w_i, w_f, w_o, b_i, b_f, b_c, b_o)

    return (o, (hn, cn))


# speedup vs baseline: 1.0903x; 1.0767x over previous
import functools

import jax
import jax.numpy as jnp
from jax import lax
from jax.experimental import pallas as pl
from jax.experimental.pallas import tpu as pltpu
from jax.experimental.pallas import tpu_sc as plsc

NC = 2
NS = 16
CH = 128


def _make_seg_sum(npad, d, epad, f0=0.5):
    nw = NC * NS
    total_chunks = epad // CH
    nch0 = int(round(total_chunks * f0 / (NS * 8))) * 8
    nch0 = max(8, min(nch0, total_chunks // NS - 8))
    nch1 = total_chunks // NS - nch0
    c0_rows = NS * nch0
    rps = npad // NS
    mesh = plsc.VectorSubcoreMesh(core_axis_name="c", subcore_axis_name="s")

    @functools.partial(
        pl.kernel,
        mesh=mesh,
        out_type=[
            jax.ShapeDtypeStruct((NC, npad, d), jnp.float32),
            jax.ShapeDtypeStruct((NC, npad, d), jnp.float32),
            jax.ShapeDtypeStruct((NC, npad, d), jnp.float32),
        ],
        scratch_types=[
            pltpu.VMEM_SHARED((npad, d), jnp.float32),
            pltpu.VMEM((8, CH), jnp.int32),
            pltpu.VMEM((8, CH), jnp.int32),
            pltpu.VMEM((2, CH, d), jnp.float32),
            pltpu.SemaphoreType.DMA,
            pltpu.SemaphoreType.DMA,
            pltpu.SemaphoreType.DMA,
            pltpu.SemaphoreType.DMA,
        ],
    )
    def seg_sum(x_hbm, h_hbm, src_hbm, dst_hbm, zr_hbm, on_hbm,
                px_hbm, ph_hbm, pc_hbm,
                acc, src_v, dst_v, rows_v, gs0, gs1, ss0, ss1):
        cid = lax.axis_index("c")
        sid = lax.axis_index("s")
        own = pl.ds(sid * rps, rps)
        gsem = (gs0, gs1)
        ssem = (ss0, ss1)
        wbase = jnp.where(cid == 0, sid * nch0, c0_rows + sid * nch1)
        ngroups = jnp.where(cid == 0, nch0 // 8, nch1 // 8)

        def zero_acc():
            pltpu.sync_copy(zr_hbm, acc.at[own])

        def dump(out_hbm):
            pltpu.sync_copy(acc.at[own], out_hbm.at[cid].at[own])

        def cnt_phase():
            pltpu.sync_copy(on_hbm, rows_v.at[0])

            def group(g, carry):
                base = wbase + g * 8
                pltpu.sync_copy(dst_hbm.at[pl.ds(base, 8)], dst_v)
                sca = []
                for k in range(8):
                    if k >= 2:
                        sca[k - 2].wait()
                    sca.append(pltpu.async_copy(
                        rows_v.at[0], acc.at[dst_v.at[k]],
                        ssem[k % 2], add=True))
                sca[6].wait()
                sca[7].wait()
                return carry

            lax.fori_loop(0, ngroups, group, 0)

        def agg_phase(table_hbm):
            def group(g, carry):
                base = wbase + g * 8
                pltpu.sync_copy(src_hbm.at[pl.ds(base, 8)], src_v)
                pltpu.sync_copy(dst_hbm.at[pl.ds(base, 8)], dst_v)
                gat = [pltpu.async_copy(
                    table_hbm.at[src_v.at[0]], rows_v.at[0], gsem[0])]
                sca = []
                for k in range(8):
                    gat[k].wait()
                    sca.append(pltpu.async_copy(
                        rows_v.at[k % 2], acc.at[dst_v.at[k]],
                        ssem[k % 2], add=True))
                    if k < 7:
                        if k >= 1:
                            sca[k - 1].wait()
                        gat.append(pltpu.async_copy(
                            table_hbm.at[src_v.at[k + 1]],
                            rows_v.at[(k + 1) % 2], gsem[(k + 1) % 2]))
                sca[6].wait()
                sca[7].wait()
                return carry

            lax.fori_loop(0, ngroups, group, 0)

        zero_acc()
        plsc.subcore_barrier()
        cnt_phase()
        plsc.subcore_barrier()
        dump(pc_hbm)

        zero_acc()
        plsc.subcore_barrier()
        agg_phase(x_hbm)
        plsc.subcore_barrier()
        dump(px_hbm)

        zero_acc()
        plsc.subcore_barrier()
        agg_phase(h_hbm)
        plsc.subcore_barrier()
        dump(ph_hbm)

    return seg_sum


def _tc_body(px, ph, pc, x, h, c, w, b, wi, wf, wo, bi, bf, bc, bo,
             o_ref, hn_ref, cn_ref):
    sx = px[0] + px[1]
    sh = ph[0] + ph[1]
    cnt = jnp.maximum(pc[0][:, 0:1] + pc[1][:, 0:1], 1.0)
    agg_x = sx / cnt
    agg_h = sh / cnt
    a = jnp.concatenate([agg_x, x[...], agg_h, h[...]], axis=1)
    z = jnp.dot(a, w[...], preferred_element_type=jnp.float32) + b[...]
    hd = x.shape[1]
    c0 = c[...]
    zi = z[:, 0 * hd:1 * hd]
    zf = z[:, 1 * hd:2 * hd]
    zc = z[:, 2 * hd:3 * hd]
    zo = z[:, 3 * hd:4 * hd]
    gi = jax.nn.sigmoid(zi + wi[...] * c0 + bi[...])
    gf = jax.nn.sigmoid(zf + wf[...] * c0 + bf[...])
    cn = gf * c0 + gi * jnp.tanh(zc + bc[...])
    go = jax.nn.sigmoid(zo + wo[...] * cn + bo[...])
    o_ref[...] = go
    hn_ref[...] = go * jnp.tanh(cn)
    cn_ref[...] = cn


def kernel(X, edge_index, h, c,
           ix_Wl, ix_bl, ix_Wr, ih_Wl, ih_bl, ih_Wr,
           fx_Wl, fx_bl, fx_Wr, fh_Wl, fh_bl, fh_Wr,
           cx_Wl, cx_bl, cx_Wr, ch_Wl, ch_bl, ch_Wr,
           ox_Wl, ox_bl, ox_Wr, oh_Wl, oh_bl, oh_Wr,
           w_i, w_f, w_o, b_i, b_f, b_c, b_o):
    n, d = X.shape
    e = edge_index.shape[1]
    hd = h.shape[1]
    nw = NC * NS
    npad = -(-n // (8 * NS)) * (8 * NS)
    rps = npad // NS
    estep = nw * CH * 8
    epad = -(-e // estep) * estep

    src = edge_index[0]
    dst = edge_index[1]
    if epad != e:
        src = jnp.concatenate([src, jnp.zeros((epad - e,), jnp.int32)])
        dst = jnp.concatenate(
            [dst, jnp.full((epad - e,), npad - 1, jnp.int32)])
    srcm = src.reshape(-1, CH)
    dstm = dst.reshape(-1, CH)
    zr = jnp.zeros((rps, d), jnp.float32)
    on = jnp.ones((CH, d), jnp.float32)

    px, ph, pc = _make_seg_sum(npad, d, epad, f0=0.90)(X, h, srcm, dstm, zr, on)

    wcols = []
    for wl_x, wr_x, wl_h, wr_h in (
        (ix_Wl, ix_Wr, ih_Wl, ih_Wr),
        (fx_Wl, fx_Wr, fh_Wl, fh_Wr),
        (cx_Wl, cx_Wr, ch_Wl, ch_Wr),
        (ox_Wl, ox_Wr, oh_Wl, oh_Wr),
    ):
        wcols.append(jnp.concatenate([wl_x.T, wr_x.T, wl_h.T, wr_h.T], axis=0))
    wstack = jnp.concatenate(wcols, axis=1)
    bias = jnp.concatenate(
        [ix_bl + ih_bl, fx_bl + fh_bl, cx_bl + ch_bl, ox_bl + oh_bl]
    ).reshape(1, 4 * hd)

    blk = 400
    grid = (n // blk,)
    full = lambda i: (0, 0)
    row = lambda i: (i, 0)
    part3 = lambda i: (0, i, 0)
    part2 = lambda i: (0, i)
    o, hn, cn = pl.pallas_call(
        _tc_body,
        grid=grid,
        in_specs=[
            pl.BlockSpec((NC, blk, d), part3),
            pl.BlockSpec((NC, blk, d), part3),
            pl.BlockSpec((NC, blk, d), part3),
            pl.BlockSpec((blk, d), row),
            pl.BlockSpec((blk, hd), row),
            pl.BlockSpec((blk, hd), row),
            pl.BlockSpec((2 * d + 2 * hd, 4 * hd), full),
            pl.BlockSpec((1, 4 * hd), full),
            pl.BlockSpec((1, hd), full),
            pl.BlockSpec((1, hd), full),
            pl.BlockSpec((1, hd), full),
            pl.BlockSpec((1, hd), full),
            pl.BlockSpec((1, hd), full),
            pl.BlockSpec((1, hd), full),
            pl.BlockSpec((1, hd), full),
        ],
        out_specs=[
            pl.BlockSpec((blk, hd), row),
            pl.BlockSpec((blk, hd), row),
            pl.BlockSpec((blk, hd), row),
        ],
        out_shape=[
            jax.ShapeDtypeStruct((n, hd), jnp.float32),
            jax.ShapeDtypeStruct((n, hd), jnp.float32),
            jax.ShapeDtypeStruct((n, hd), jnp.float32),
        ],
    )(px, ph, pc, X, h, c, wstack, bias, w_i, w_f, w_o, b_i, b_f, b_c, b_o)

    return (o, (hn, cn))
